# Initial kernel scaffold; baseline (speedup 1.0000x reference)
import functools

import jax
import jax.numpy as jnp
from jax import lax
from jax.experimental import pallas as pl
from jax.experimental.pallas import tpu as pltpu
from jax.experimental.pallas import tpu_sc as plsc

N = 10000
E = 320000
D_IN = 128
H1, H2, H3 = 128, 64, 32

NC = 2
NS = 16
NW = NC * NS
EPW = E // NW
K = 80
NCH = EPW // K
RPT = N // NS
DEGW = 16

_BN_EPS = 1e-5
_BR = 2000


def _make_propagate(D):
    mesh = plsc.VectorSubcoreMesh(core_axis_name="c", subcore_axis_name="s")

    @functools.partial(
        pl.kernel,
        out_type=jax.ShapeDtypeStruct((NC, N, D), jnp.float32),
        mesh=mesh,
        scratch_types=[
            pltpu.VMEM((NCH, K), jnp.int32),
            pltpu.VMEM((NCH, K), jnp.int32),
            pltpu.VMEM((K, D), jnp.float32),
            pltpu.VMEM_SHARED((N, D), jnp.float32),
            pltpu.SemaphoreType.DMA,
        ],
    )
    def prop(src_hbm, dst_hbm, y_hbm, zeros_hbm, out_hbm,
             src_v, dst_v, rows_v, acc_sh, sem):
        c = lax.axis_index("c")
        s = lax.axis_index("s")
        wid = s * NC + c
        pltpu.sync_copy(src_hbm.at[wid], src_v)
        pltpu.sync_copy(dst_hbm.at[wid], dst_v)
        pltpu.sync_copy(zeros_hbm, acc_sh.at[pl.ds(s * RPT, RPT)])
        plsc.subcore_barrier()

        def body(j, carry):
            pltpu.async_copy(y_hbm.at[src_v.at[j]], rows_v, sem).wait()
            pltpu.sync_copy(rows_v, acc_sh.at[dst_v.at[j]], add=True)
            return carry

        lax.fori_loop(0, NCH, body, 0)
        plsc.subcore_barrier()
        pltpu.sync_copy(acc_sh.at[pl.ds(s * RPT, RPT)],
                        out_hbm.at[c, pl.ds(s * RPT, RPT)])

    return prop


_propagate = {D: _make_propagate(D) for D in (H1, H2, H3)}

_deg_mesh = plsc.VectorSubcoreMesh(core_axis_name="c", subcore_axis_name="s")


@functools.partial(
    pl.kernel,
    out_type=jax.ShapeDtypeStruct((NC, N, DEGW), jnp.float32),
    mesh=_deg_mesh,
    scratch_types=[
        pltpu.VMEM((NCH, K), jnp.int32),
        pltpu.VMEM((K, DEGW), jnp.float32),
        pltpu.VMEM_SHARED((N, DEGW), jnp.float32),
        pltpu.SemaphoreType.DMA,
    ],
)
def _deg_kernel(dst_hbm, ones_hbm, zeros_hbm, out_hbm,
                dst_v, ones_v, acc_sh, sem):
    c = lax.axis_index("c")
    s = lax.axis_index("s")
    wid = s * NC + c
    pltpu.sync_copy(dst_hbm.at[wid], dst_v)
    pltpu.sync_copy(ones_hbm, ones_v)
    pltpu.sync_copy(zeros_hbm, acc_sh.at[pl.ds(s * RPT, RPT)])
    plsc.subcore_barrier()

    def body(j, carry):
        pltpu.sync_copy(ones_v, acc_sh.at[dst_v.at[j]], add=True)
        return carry

    lax.fori_loop(0, NCH, body, 0)
    plsc.subcore_barrier()
    pltpu.sync_copy(acc_sh.at[pl.ds(s * RPT, RPT)],
                    out_hbm.at[c, pl.ds(s * RPT, RPT)])


def _dinv_body(dp_ref, o_ref):
    deg = dp_ref[0, :, 0:1] + dp_ref[1, :, 0:1] + 1.0
    o_ref[...] = lax.rsqrt(deg)


def _dinv(degp):
    grid = N // _BR
    return pl.pallas_call(
        _dinv_body,
        grid=(grid,),
        in_specs=[pl.BlockSpec((NC, _BR, DEGW), lambda i: (0, i, 0))],
        out_specs=pl.BlockSpec((_BR, 1), lambda i: (i, 0)),
        out_shape=jax.ShapeDtypeStruct((N, 1), jnp.float32),
    )(degp)


def _mm_scale_body(x_ref, w_ref, dinv_ref, o_ref):
    o_ref[...] = jnp.dot(x_ref[...], w_ref[...],
                         preferred_element_type=jnp.float32) * dinv_ref[...]


def _mm_scale(x, w, dinv):
    din, dout = w.shape
    grid = N // _BR
    return pl.pallas_call(
        _mm_scale_body,
        grid=(grid,),
        in_specs=[
            pl.BlockSpec((_BR, din), lambda i: (i, 0)),
            pl.BlockSpec((din, dout), lambda i: (0, 0)),
            pl.BlockSpec((_BR, 1), lambda i: (i, 0)),
        ],
        out_specs=pl.BlockSpec((_BR, dout), lambda i: (i, 0)),
        out_shape=jax.ShapeDtypeStruct((N, dout), jnp.float32),
    )(x, w, dinv)


def _comb_body(p_ref, y_ref, dinv_ref, o_ref, s_ref):
    h = (p_ref[0] + p_ref[1] + y_ref[...]) * dinv_ref[...]
    o_ref[...] = h

    @pl.when(pl.program_id(0) == 0)
    def _init():
        s_ref[...] = jnp.zeros_like(s_ref)

    s0 = jnp.sum(h, axis=0, keepdims=True)
    s1 = jnp.sum(h * h, axis=0, keepdims=True)
    s_ref[...] += jnp.concatenate([s0, s1], axis=0)


def _comb(p, y, dinv):
    d = y.shape[1]
    grid = N // _BR
    return pl.pallas_call(
        _comb_body,
        grid=(grid,),
        in_specs=[
            pl.BlockSpec((NC, _BR, d), lambda i: (0, i, 0)),
            pl.BlockSpec((_BR, d), lambda i: (i, 0)),
            pl.BlockSpec((_BR, 1), lambda i: (i, 0)),
        ],
        out_specs=[
            pl.BlockSpec((_BR, d), lambda i: (i, 0)),
            pl.BlockSpec((2, d), lambda i: (0, 0)),
        ],
        out_shape=[
            jax.ShapeDtypeStruct((N, d), jnp.float32),
            jax.ShapeDtypeStruct((2, d), jnp.float32),
        ],
    )(p, y, dinv)


def _bn_mm_body(h_ref, s_ref, g_ref, be_ref, w_ref, b_ref, dinv_ref, o_ref):
    mean = s_ref[0:1, :] * (1.0 / N)
    var = s_ref[1:2, :] * (1.0 / N) - mean * mean
    scale = g_ref[...] * lax.rsqrt(var + _BN_EPS)
    shift = be_ref[...] - mean * scale
    h = jnp.maximum(h_ref[...] * scale + shift, 0.0)
    o_ref[...] = (jnp.dot(h, w_ref[...], preferred_element_type=jnp.float32)
                  + b_ref[...]) * dinv_ref[...]


def _bn_mm(h, stats, g, be, w, b, dinv):
    din, dout = w.shape
    grid = N // _BR
    return pl.pallas_call(
        _bn_mm_body,
        grid=(grid,),
        in_specs=[
            pl.BlockSpec((_BR, din), lambda i: (i, 0)),
            pl.BlockSpec((2, din), lambda i: (0, 0)),
            pl.BlockSpec((1, din), lambda i: (0, 0)),
            pl.BlockSpec((1, din), lambda i: (0, 0)),
            pl.BlockSpec((din, dout), lambda i: (0, 0)),
            pl.BlockSpec((1, dout), lambda i: (0, 0)),
            pl.BlockSpec((_BR, 1), lambda i: (i, 0)),
        ],
        out_specs=pl.BlockSpec((_BR, dout), lambda i: (i, 0)),
        out_shape=jax.ShapeDtypeStruct((N, dout), jnp.float32),
    )(h, stats, g, be, w, b, dinv)


def kernel(x, edge_index, W1, b1, g1, be1, W2, b2, g2, be2, W3, b3, g3, be3,
           Wf, bf):
    src = edge_index[0].reshape(NW, NCH, K).astype(jnp.int32)
    dst = edge_index[1].reshape(NW, NCH, K).astype(jnp.int32)

    ones_deg = jnp.ones((K, DEGW), jnp.float32)
    zeros_deg = jnp.zeros((RPT, DEGW), jnp.float32)
    degp = _deg_kernel(dst, ones_deg, zeros_deg)
    dinv = _dinv(degp)

    g1r, be1r = g1.reshape(1, H1), be1.reshape(1, H1)
    g2r, be2r = g2.reshape(1, H2), be2.reshape(1, H2)
    g3r, be3r = g3.reshape(1, H3), be3.reshape(1, H3)
    wf_pad = jnp.zeros((H3, 128), jnp.float32).at[:, 0:1].set(Wf)
    bf_pad = jnp.zeros((1, 128), jnp.float32).at[0, 0].set(bf[0])
    zeros2 = jnp.zeros((1, H2), jnp.float32)
    zeros3 = jnp.zeros((1, H3), jnp.float32)
    ones_n = jnp.ones((N, 1), jnp.float32)

    y1 = _mm_scale(x, W1, dinv)
    p1 = _propagate[H1](src, dst, y1, jnp.zeros((RPT, H1), jnp.float32))
    h1, s1 = _comb(p1, y1, dinv)
    y2 = _bn_mm(h1, s1, g1r, be1r, W2, zeros2, dinv)
    p2 = _propagate[H2](src, dst, y2, jnp.zeros((RPT, H2), jnp.float32))
    h2, s2 = _comb(p2, y2, dinv)
    y3 = _bn_mm(h2, s2, g2r, be2r, W3, zeros3, dinv)
    p3 = _propagate[H3](src, dst, y3, jnp.zeros((RPT, H3), jnp.float32))
    h3, s3 = _comb(p3, y3, dinv)
    out = _bn_mm(h3, s3, g3r, be3r, wf_pad, bf_pad, ones_n)
    return out[:, 0:1]


# speedup vs baseline: 19.5290x; 19.5290x over previous
import functools

import jax
import jax.numpy as jnp
from jax import lax
from jax.experimental import pallas as pl
from jax.experimental.pallas import tpu as pltpu
from jax.experimental.pallas import tpu_sc as plsc

N = 10000
E = 320000
D_IN = 128
H1, H2, H3 = 128, 64, 32

NC = 2
NS = 16
NW = NC * NS
EPW = E // NW
K = 80
NCH = EPW // K
NP = 10240
RPT = NP // NS
DEGW = 16

_BN_EPS = 1e-5
_BR = 2000


def _make_propagate(D):
    mesh = plsc.VectorSubcoreMesh(core_axis_name="c", subcore_axis_name="s")

    @functools.partial(
        pl.kernel,
        out_type=jax.ShapeDtypeStruct((NC, NP, D), jnp.float32),
        mesh=mesh,
        scratch_types=[
            pltpu.VMEM((NCH, K), jnp.int32),
            pltpu.VMEM((NCH, K), jnp.int32),
            pltpu.VMEM((K, D), jnp.float32),
            pltpu.VMEM_SHARED((NP, D), jnp.float32),
            pltpu.SemaphoreType.DMA,
        ],
        compiler_params=pltpu.CompilerParams(use_tc_tiling_on_sc=False),
    )
    def prop(src_hbm, dst_hbm, y_hbm, zeros_hbm, out_hbm,
             src_v, dst_v, rows_v, acc_sh, sem):
        c = lax.axis_index("c")
        s = lax.axis_index("s")
        wid = s * NC + c
        pltpu.sync_copy(src_hbm.at[wid], src_v)
        pltpu.sync_copy(dst_hbm.at[wid], dst_v)
        pltpu.sync_copy(zeros_hbm, acc_sh.at[pl.ds(s * RPT, RPT)])
        plsc.subcore_barrier()

        def body(j, carry):
            pltpu.async_copy(y_hbm.at[src_v.at[j]], rows_v, sem).wait()
            pltpu.sync_copy(rows_v, acc_sh.at[dst_v.at[j]], add=True)
            return carry

        lax.fori_loop(0, NCH, body, 0)
        plsc.subcore_barrier()
        pltpu.sync_copy(acc_sh.at[pl.ds(s * RPT, RPT)],
                        out_hbm.at[c, pl.ds(s * RPT, RPT)])

    return prop


_propagate = {D: _make_propagate(D) for D in (H1, H2, H3)}

_deg_mesh = plsc.VectorSubcoreMesh(core_axis_name="c", subcore_axis_name="s")


@functools.partial(
    pl.kernel,
    out_type=jax.ShapeDtypeStruct((NC, NP, DEGW), jnp.float32),
    mesh=_deg_mesh,
    scratch_types=[
        pltpu.VMEM((NCH, K), jnp.int32),
        pltpu.VMEM((K, DEGW), jnp.float32),
        pltpu.VMEM_SHARED((NP, DEGW), jnp.float32),
        pltpu.SemaphoreType.DMA,
    ],
    compiler_params=pltpu.CompilerParams(use_tc_tiling_on_sc=False),
)
def _deg_kernel(dst_hbm, ones_hbm, zeros_hbm, out_hbm,
                dst_v, ones_v, acc_sh, sem):
    c = lax.axis_index("c")
    s = lax.axis_index("s")
    wid = s * NC + c
    pltpu.sync_copy(dst_hbm.at[wid], dst_v)
    pltpu.sync_copy(ones_hbm, ones_v)
    pltpu.sync_copy(zeros_hbm, acc_sh.at[pl.ds(s * RPT, RPT)])
    plsc.subcore_barrier()

    def body(j, carry):
        pltpu.sync_copy(ones_v, acc_sh.at[dst_v.at[j]], add=True)
        return carry

    lax.fori_loop(0, NCH, body, 0)
    plsc.subcore_barrier()
    pltpu.sync_copy(acc_sh.at[pl.ds(s * RPT, RPT)],
                    out_hbm.at[c, pl.ds(s * RPT, RPT)])


def _dinv_body(dp_ref, o_ref):
    deg = dp_ref[0, :, 0:1] + dp_ref[1, :, 0:1] + 1.0
    o_ref[...] = lax.rsqrt(deg)


def _dinv(degp):
    grid = N // _BR
    return pl.pallas_call(
        _dinv_body,
        grid=(grid,),
        in_specs=[pl.BlockSpec((NC, _BR, DEGW), lambda i: (0, i, 0))],
        out_specs=pl.BlockSpec((_BR, 1), lambda i: (i, 0)),
        out_shape=jax.ShapeDtypeStruct((N, 1), jnp.float32),
    )(degp)


def _mm_scale_body(x_ref, w_ref, dinv_ref, o_ref):
    o_ref[...] = jnp.dot(x_ref[...], w_ref[...],
                         preferred_element_type=jnp.float32) * dinv_ref[...]


def _mm_scale(x, w, dinv):
    din, dout = w.shape
    grid = N // _BR
    return pl.pallas_call(
        _mm_scale_body,
        grid=(grid,),
        in_specs=[
            pl.BlockSpec((_BR, din), lambda i: (i, 0)),
            pl.BlockSpec((din, dout), lambda i: (0, 0)),
            pl.BlockSpec((_BR, 1), lambda i: (i, 0)),
        ],
        out_specs=pl.BlockSpec((_BR, dout), lambda i: (i, 0)),
        out_shape=jax.ShapeDtypeStruct((N, dout), jnp.float32),
    )(x, w, dinv)


def _comb_body(p_ref, y_ref, dinv_ref, o_ref, s_ref):
    h = (p_ref[0] + p_ref[1] + y_ref[...]) * dinv_ref[...]
    o_ref[...] = h

    @pl.when(pl.program_id(0) == 0)
    def _init():
        s_ref[...] = jnp.zeros_like(s_ref)

    s0 = jnp.sum(h, axis=0, keepdims=True)
    s1 = jnp.sum(h * h, axis=0, keepdims=True)
    s_ref[...] += jnp.concatenate([s0, s1], axis=0)


def _comb(p, y, dinv):
    d = y.shape[1]
    grid = N // _BR
    return pl.pallas_call(
        _comb_body,
        grid=(grid,),
        in_specs=[
            pl.BlockSpec((NC, _BR, d), lambda i: (0, i, 0)),
            pl.BlockSpec((_BR, d), lambda i: (i, 0)),
            pl.BlockSpec((_BR, 1), lambda i: (i, 0)),
        ],
        out_specs=[
            pl.BlockSpec((_BR, d), lambda i: (i, 0)),
            pl.BlockSpec((2, d), lambda i: (0, 0)),
        ],
        out_shape=[
            jax.ShapeDtypeStruct((N, d), jnp.float32),
            jax.ShapeDtypeStruct((2, d), jnp.float32),
        ],
    )(p, y, dinv)


def _bn_mm_body(h_ref, s_ref, g_ref, be_ref, w_ref, b_ref, dinv_ref, o_ref):
    mean = s_ref[0:1, :] * (1.0 / N)
    var = s_ref[1:2, :] * (1.0 / N) - mean * mean
    scale = g_ref[...] * lax.rsqrt(var + _BN_EPS)
    shift = be_ref[...] - mean * scale
    h = jnp.maximum(h_ref[...] * scale + shift, 0.0)
    o_ref[...] = (jnp.dot(h, w_ref[...], preferred_element_type=jnp.float32)
                  + b_ref[...]) * dinv_ref[...]


def _bn_mm(h, stats, g, be, w, b, dinv):
    din, dout = w.shape
    grid = N // _BR
    return pl.pallas_call(
        _bn_mm_body,
        grid=(grid,),
        in_specs=[
            pl.BlockSpec((_BR, din), lambda i: (i, 0)),
            pl.BlockSpec((2, din), lambda i: (0, 0)),
            pl.BlockSpec((1, din), lambda i: (0, 0)),
            pl.BlockSpec((1, din), lambda i: (0, 0)),
            pl.BlockSpec((din, dout), lambda i: (0, 0)),
            pl.BlockSpec((1, dout), lambda i: (0, 0)),
            pl.BlockSpec((_BR, 1), lambda i: (i, 0)),
        ],
        out_specs=pl.BlockSpec((_BR, dout), lambda i: (i, 0)),
        out_shape=jax.ShapeDtypeStruct((N, dout), jnp.float32),
    )(h, stats, g, be, w, b, dinv)


def kernel(x, edge_index, W1, b1, g1, be1, W2, b2, g2, be2, W3, b3, g3, be3,
           Wf, bf):
    src = edge_index[0].reshape(NW, NCH, K).astype(jnp.int32)
    dst = edge_index[1].reshape(NW, NCH, K).astype(jnp.int32)

    ones_deg = jnp.ones((K, DEGW), jnp.float32)
    zeros_deg = jnp.zeros((RPT, DEGW), jnp.float32)
    degp = _deg_kernel(dst, ones_deg, zeros_deg)
    dinv = _dinv(degp)

    g1r, be1r = g1.reshape(1, H1), be1.reshape(1, H1)
    g2r, be2r = g2.reshape(1, H2), be2.reshape(1, H2)
    g3r, be3r = g3.reshape(1, H3), be3.reshape(1, H3)
    wf_pad = jnp.zeros((H3, 128), jnp.float32).at[:, 0:1].set(Wf)
    bf_pad = jnp.zeros((1, 128), jnp.float32).at[0, 0].set(bf[0])
    zeros2 = jnp.zeros((1, H2), jnp.float32)
    zeros3 = jnp.zeros((1, H3), jnp.float32)
    ones_n = jnp.ones((N, 1), jnp.float32)

    y1 = _mm_scale(x, W1, dinv)
    p1 = _propagate[H1](src, dst, y1, jnp.zeros((RPT, H1), jnp.float32))
    h1, s1 = _comb(p1, y1, dinv)
    y2 = _bn_mm(h1, s1, g1r, be1r, W2, zeros2, dinv)
    p2 = _propagate[H2](src, dst, y2, jnp.zeros((RPT, H2), jnp.float32))
    h2, s2 = _comb(p2, y2, dinv)
    y3 = _bn_mm(h2, s2, g2r, be2r, W3, zeros3, dinv)
    p3 = _propagate[H3](src, dst, y3, jnp.zeros((RPT, H3), jnp.float32))
    h3, s3 = _comb(p3, y3, dinv)
    out = _bn_mm(h3, s3, g3r, be3r, wf_pad, bf_pad, ones_n)
    return out[:, 0:1]


# speedup vs baseline: 28.5755x; 1.4632x over previous
import functools

import jax
import jax.numpy as jnp
from jax import lax
from jax.experimental import pallas as pl
from jax.experimental.pallas import tpu as pltpu
from jax.experimental.pallas import tpu_sc as plsc

N = 10000
E = 320000
D_IN = 128
H1, H2, H3 = 128, 64, 32

NC = 2
NS = 16
NW = NC * NS
EPW = E // NW
K = 80
NCH = EPW // K
NP = 10240
RPT = NP // NS
DEGW = 16

_BN_EPS = 1e-5
_BR = 2000


def _make_propagate(D):
    mesh = plsc.VectorSubcoreMesh(core_axis_name="c", subcore_axis_name="s")

    @functools.partial(
        pl.kernel,
        out_type=jax.ShapeDtypeStruct((NC, NP, D), jnp.float32),
        mesh=mesh,
        scratch_types=[
            pltpu.VMEM((NCH, K), jnp.int32),
            pltpu.VMEM((NCH, K), jnp.int32),
            pltpu.VMEM((K, D), jnp.float32),
            pltpu.VMEM((K, D), jnp.float32),
            pltpu.VMEM_SHARED((NP, D), jnp.float32),
            pltpu.SemaphoreType.DMA,
            pltpu.SemaphoreType.DMA,
        ],
        compiler_params=pltpu.CompilerParams(use_tc_tiling_on_sc=False),
    )
    def prop(src_hbm, dst_hbm, y_hbm, zeros_hbm, out_hbm,
             src_v, dst_v, rows_a, rows_b, acc_sh, sem_a, sem_b):
        c = lax.axis_index("c")
        s = lax.axis_index("s")
        wid = s * NC + c
        pltpu.sync_copy(src_hbm.at[wid], src_v)
        pltpu.sync_copy(dst_hbm.at[wid], dst_v)
        pltpu.sync_copy(zeros_hbm, acc_sh.at[pl.ds(s * RPT, RPT)])
        plsc.subcore_barrier()

        pltpu.async_copy(y_hbm.at[src_v.at[0]], rows_a, sem_a)

        def body(i, carry):
            j0 = 2 * i
            pltpu.async_copy(y_hbm.at[src_v.at[j0 + 1]], rows_b, sem_b)
            pltpu.make_async_copy(y_hbm.at[src_v.at[j0]], rows_a, sem_a).wait()
            pltpu.sync_copy(rows_a, acc_sh.at[dst_v.at[j0]], add=True)
            pltpu.async_copy(y_hbm.at[src_v.at[j0 + 2]], rows_a, sem_a)
            pltpu.make_async_copy(y_hbm.at[src_v.at[j0 + 1]], rows_b, sem_b).wait()
            pltpu.sync_copy(rows_b, acc_sh.at[dst_v.at[j0 + 1]], add=True)
            return carry

        lax.fori_loop(0, (NCH - 1) // 2, body, 0)
        pltpu.make_async_copy(y_hbm.at[src_v.at[NCH - 1]], rows_a, sem_a).wait()
        pltpu.sync_copy(rows_a, acc_sh.at[dst_v.at[NCH - 1]], add=True)
        plsc.subcore_barrier()
        pltpu.sync_copy(acc_sh.at[pl.ds(s * RPT, RPT)],
                        out_hbm.at[c, pl.ds(s * RPT, RPT)])

    return prop


_propagate = {D: _make_propagate(D) for D in (H1, H2, H3)}

_deg_mesh = plsc.VectorSubcoreMesh(core_axis_name="c", subcore_axis_name="s")


@functools.partial(
    pl.kernel,
    out_type=jax.ShapeDtypeStruct((NC, NP, DEGW), jnp.float32),
    mesh=_deg_mesh,
    scratch_types=[
        pltpu.VMEM((NCH, K), jnp.int32),
        pltpu.VMEM((K, DEGW), jnp.float32),
        pltpu.VMEM_SHARED((NP, DEGW), jnp.float32),
        pltpu.SemaphoreType.DMA,
    ],
    compiler_params=pltpu.CompilerParams(use_tc_tiling_on_sc=False),
)
def _deg_kernel(dst_hbm, ones_hbm, zeros_hbm, out_hbm,
                dst_v, ones_v, acc_sh, sem):
    c = lax.axis_index("c")
    s = lax.axis_index("s")
    wid = s * NC + c
    pltpu.sync_copy(dst_hbm.at[wid], dst_v)
    pltpu.sync_copy(ones_hbm, ones_v)
    pltpu.sync_copy(zeros_hbm, acc_sh.at[pl.ds(s * RPT, RPT)])
    plsc.subcore_barrier()

    def body(j, carry):
        pltpu.sync_copy(ones_v, acc_sh.at[dst_v.at[j]], add=True)
        return carry

    lax.fori_loop(0, NCH, body, 0)
    plsc.subcore_barrier()
    pltpu.sync_copy(acc_sh.at[pl.ds(s * RPT, RPT)],
                    out_hbm.at[c, pl.ds(s * RPT, RPT)])


def _dinv_body(dp_ref, o_ref):
    deg = dp_ref[0, :, 0:1] + dp_ref[1, :, 0:1] + 1.0
    o_ref[...] = lax.rsqrt(deg)


def _dinv(degp):
    grid = N // _BR
    return pl.pallas_call(
        _dinv_body,
        grid=(grid,),
        in_specs=[pl.BlockSpec((NC, _BR, DEGW), lambda i: (0, i, 0))],
        out_specs=pl.BlockSpec((_BR, 1), lambda i: (i, 0)),
        out_shape=jax.ShapeDtypeStruct((N, 1), jnp.float32),
    )(degp)


def _mm_scale_body(x_ref, w_ref, dinv_ref, o_ref):
    o_ref[...] = jnp.dot(x_ref[...], w_ref[...],
                         preferred_element_type=jnp.float32) * dinv_ref[...]


def _mm_scale(x, w, dinv):
    din, dout = w.shape
    grid = N // _BR
    return pl.pallas_call(
        _mm_scale_body,
        grid=(grid,),
        in_specs=[
            pl.BlockSpec((_BR, din), lambda i: (i, 0)),
            pl.BlockSpec((din, dout), lambda i: (0, 0)),
            pl.BlockSpec((_BR, 1), lambda i: (i, 0)),
        ],
        out_specs=pl.BlockSpec((_BR, dout), lambda i: (i, 0)),
        out_shape=jax.ShapeDtypeStruct((N, dout), jnp.float32),
    )(x, w, dinv)


def _comb_body(p_ref, y_ref, dinv_ref, o_ref, s_ref):
    h = (p_ref[0] + p_ref[1] + y_ref[...]) * dinv_ref[...]
    o_ref[...] = h

    @pl.when(pl.program_id(0) == 0)
    def _init():
        s_ref[...] = jnp.zeros_like(s_ref)

    s0 = jnp.sum(h, axis=0, keepdims=True)
    s1 = jnp.sum(h * h, axis=0, keepdims=True)
    s_ref[...] += jnp.concatenate([s0, s1], axis=0)


def _comb(p, y, dinv):
    d = y.shape[1]
    grid = N // _BR
    return pl.pallas_call(
        _comb_body,
        grid=(grid,),
        in_specs=[
            pl.BlockSpec((NC, _BR, d), lambda i: (0, i, 0)),
            pl.BlockSpec((_BR, d), lambda i: (i, 0)),
            pl.BlockSpec((_BR, 1), lambda i: (i, 0)),
        ],
        out_specs=[
            pl.BlockSpec((_BR, d), lambda i: (i, 0)),
            pl.BlockSpec((2, d), lambda i: (0, 0)),
        ],
        out_shape=[
            jax.ShapeDtypeStruct((N, d), jnp.float32),
            jax.ShapeDtypeStruct((2, d), jnp.float32),
        ],
    )(p, y, dinv)


def _bn_mm_body(h_ref, s_ref, g_ref, be_ref, w_ref, b_ref, dinv_ref, o_ref):
    mean = s_ref[0:1, :] * (1.0 / N)
    var = s_ref[1:2, :] * (1.0 / N) - mean * mean
    scale = g_ref[...] * lax.rsqrt(var + _BN_EPS)
    shift = be_ref[...] - mean * scale
    h = jnp.maximum(h_ref[...] * scale + shift, 0.0)
    o_ref[...] = (jnp.dot(h, w_ref[...], preferred_element_type=jnp.float32)
                  + b_ref[...]) * dinv_ref[...]


def _bn_mm(h, stats, g, be, w, b, dinv):
    din, dout = w.shape
    grid = N // _BR
    return pl.pallas_call(
        _bn_mm_body,
        grid=(grid,),
        in_specs=[
            pl.BlockSpec((_BR, din), lambda i: (i, 0)),
            pl.BlockSpec((2, din), lambda i: (0, 0)),
            pl.BlockSpec((1, din), lambda i: (0, 0)),
            pl.BlockSpec((1, din), lambda i: (0, 0)),
            pl.BlockSpec((din, dout), lambda i: (0, 0)),
            pl.BlockSpec((1, dout), lambda i: (0, 0)),
            pl.BlockSpec((_BR, 1), lambda i: (i, 0)),
        ],
        out_specs=pl.BlockSpec((_BR, dout), lambda i: (i, 0)),
        out_shape=jax.ShapeDtypeStruct((N, dout), jnp.float32),
    )(h, stats, g, be, w, b, dinv)


def kernel(x, edge_index, W1, b1, g1, be1, W2, b2, g2, be2, W3, b3, g3, be3,
           Wf, bf):
    src = edge_index[0].reshape(NW, NCH, K).astype(jnp.int32)
    dst = edge_index[1].reshape(NW, NCH, K).astype(jnp.int32)

    ones_deg = jnp.ones((K, DEGW), jnp.float32)
    zeros_deg = jnp.zeros((RPT, DEGW), jnp.float32)
    degp = _deg_kernel(dst, ones_deg, zeros_deg)
    dinv = _dinv(degp)

    g1r, be1r = g1.reshape(1, H1), be1.reshape(1, H1)
    g2r, be2r = g2.reshape(1, H2), be2.reshape(1, H2)
    g3r, be3r = g3.reshape(1, H3), be3.reshape(1, H3)
    wf_pad = jnp.zeros((H3, 128), jnp.float32).at[:, 0:1].set(Wf)
    bf_pad = jnp.zeros((1, 128), jnp.float32).at[0, 0].set(bf[0])
    zeros2 = jnp.zeros((1, H2), jnp.float32)
    zeros3 = jnp.zeros((1, H3), jnp.float32)
    ones_n = jnp.ones((N, 1), jnp.float32)

    y1 = _mm_scale(x, W1, dinv)
    p1 = _propagate[H1](src, dst, y1, jnp.zeros((RPT, H1), jnp.float32))
    h1, s1 = _comb(p1, y1, dinv)
    y2 = _bn_mm(h1, s1, g1r, be1r, W2, zeros2, dinv)
    p2 = _propagate[H2](src, dst, y2, jnp.zeros((RPT, H2), jnp.float32))
    h2, s2 = _comb(p2, y2, dinv)
    y3 = _bn_mm(h2, s2, g2r, be2r, W3, zeros3, dinv)
    p3 = _propagate[H3](src, dst, y3, jnp.zeros((RPT, H3), jnp.float32))
    h3, s3 = _comb(p3, y3, dinv)
    out = _bn_mm(h3, s3, g3r, be3r, wf_pad, bf_pad, ones_n)
    return out[:, 0:1]
